# baseline (speedup 1.0000x reference)
import jax
import jax.numpy as jnp
from jax.experimental import pallas as pl

_B = 1024
_V = 100000
_BM = 256
_BN = 2048


def _body(x_ref, o_ref):
    @pl.when((pl.program_id(0) == 0) & (pl.program_id(1) == 0))
    def _():
        o_ref[...] = jnp.zeros_like(o_ref)

    o_ref[...] += jnp.sum(x_ref[...], axis=0, keepdims=True).reshape(8, _BN // 8)


def kernel(cos_theta, labels):
    return pl.pallas_call(
        _body,
        out_shape=jax.ShapeDtypeStruct((8, _BN // 8), jnp.float32),
        grid=(_B // _BM, -(-_V // _BN)),
        in_specs=[pl.BlockSpec((_BM, _BN), lambda i, j: (i, j))],
        out_specs=pl.BlockSpec((8, _BN // 8), lambda i, j: (0, 0)),
    )(cos_theta)
